# baseline (speedup 1.0000x reference)
import functools

import jax
import jax.numpy as jnp
from jax import lax
from jax.experimental import pallas as pl
from jax.experimental.pallas import tpu as pltpu
from jax.experimental.pallas import tpu_sc as plsc

_NC = 2
_NS = 16
_NW = _NC * _NS
_L = 16
_D = 32
_B = 16384
_PAD = _B

_UCOLS = 7813
_ICOLS = 782
_UG = 977
_IG = 98

_CAP = 768
_GCAP = 288
_FL = 96


def _prescan(ids_hbm, idbuf, lst_id, lst_pos, lo_c, hi_c, pad_id):
    iota = lax.iota(jnp.int32, _L)
    npieces = _B // 2048

    def piece(p, cnt):
        pltpu.sync_copy(ids_hbm.at[pl.ds(p * 2048, 2048)], idbuf)

        def inner(t, cnt):
            u = idbuf[pl.ds(t * _L, _L)]
            c = lax.shift_right_logical(u, 7)
            m = (c >= lo_c) & (c < hi_c)
            pos = p * 2048 + t * _L + iota
            base = jnp.minimum(cnt, _CAP)
            plsc.store_compressed(lst_id.at[pl.ds(base, _L)], u, mask=m)
            plsc.store_compressed(lst_pos.at[pl.ds(base, _L)], pos, mask=m)
            return cnt + plsc.all_reduce_population_count(m)[0]

        return lax.fori_loop(0, 2048 // _L, inner, cnt)

    def padfill(t, _):
        lst_id[pl.ds(t * _L, _L)] = jnp.full((_L,), pad_id, jnp.int32)
        lst_pos[pl.ds(t * _L, _L)] = jnp.full((_L,), _PAD, jnp.int32)
        return _

    lax.fori_loop(0, (_CAP + _L) // _L, padfill, 0)
    return lax.fori_loop(0, npieces, piece, jnp.int32(0))


def _u_body(uids_hbm, utab_hbm, scr_hbm,
            idbuf, ulist, uplist, sec, tmpu, tmpp, rows, posc,
            sem_s, sem_f):
    iota = lax.iota(jnp.int32, _L)
    wid = lax.axis_index("s") * _NC + lax.axis_index("c")
    lo_g = lax.shift_right_logical(_UG * wid, 5)
    hi_g = lax.shift_right_logical(_UG * (wid + 1), 5)

    cnt = _prescan(uids_hbm, idbuf, ulist, uplist,
                   lo_g * 8, hi_g * 8, jnp.int32(0x7FFFFFF))

    def group(g, carry):
        off = jnp.minimum(g * 8, _UCOLS - 8)
        cps = [pltpu.async_copy(
                   utab_hbm.at[pl.ds(8 * i, 8), pl.ds(off * 128, 1024)],
                   sec.at[pl.ds(8 * i, 8), :], sem_s)
               for i in range(4)]
        for c in cps:
            c.wait()

        def pad_t(t, _):
            tmpp[pl.ds(t * _L, _L)] = jnp.full((_L,), _PAD, jnp.int32)
            return _
        lax.fori_loop(0, (_FL + _L) // _L, pad_t, 0)

        def match_t(t, mc):
            u = ulist[pl.ds(t * _L, _L)]
            pv = uplist[pl.ds(t * _L, _L)]
            c = lax.shift_right_logical(u, 7)
            m = (c >= off) & (c < off + 8)
            base = jnp.minimum(mc, _FL)
            plsc.store_compressed(tmpu.at[pl.ds(base, _L)], u, mask=m)
            plsc.store_compressed(tmpp.at[pl.ds(base, _L)], pv, mask=m)
            return mc + plsc.all_reduce_population_count(m)[0]

        mc = lax.fori_loop(0, _CAP // _L, match_t, jnp.int32(0))
        mc = jnp.minimum(mc, _FL)

        for mt in range(_FL // _L):
            mu = tmpu[pl.ds(mt * _L, _L)]
            mm = (mt * _L + iota) < mc
            colv = ((lax.shift_right_logical(mu, 7) - off) * 128
                    + (mu & 127)) & 1023
            mrow = mt * _L + iota
            for d in range(_D):
                dv = jnp.full((_L,), d, jnp.int32)
                val = plsc.load_gather(sec, [dv, colv], mask=mm)
                plsc.store_scatter(rows, [mrow, dv], val, mask=mm)

        def cp_t(t, _):
            posc[pl.ds(t * _L, _L)] = tmpp[pl.ds(t * _L, _L)]
            return _
        lax.fori_loop(0, _FL // _L, cp_t, 0)
        pltpu.async_copy(rows, scr_hbm.at[posc], sem_f).wait()
        return carry

    lax.fori_loop(lo_g, hi_g, group, 0)


def _i_body(iids_hbm, itab_hbm, scr_hbm, out_hbm,
            idbuf, ilist, iplist, sec, tmpu, tmpp, irow, plist2, urows,
            outst, sem_s, sem_g, sem_o):
    iota = lax.iota(jnp.int32, _L)
    wid = lax.axis_index("s") * _NC + lax.axis_index("c")
    lo_g = lax.shift_right_logical(_IG * wid, 5)
    hi_g = lax.shift_right_logical(_IG * (wid + 1), 5)

    cnt = _prescan(iids_hbm, idbuf, ilist, iplist,
                   lo_g * 8, hi_g * 8, jnp.int32(0x7FFFFFF))

    def padp(t, _):
        r = t // (128 // _L)
        s = (t % (128 // _L)) * _L
        plsc.store_scatter(
            plist2, [jnp.full((_L,), r, jnp.int32), s + iota],
            jnp.full((_L,), _PAD, jnp.int32))
        return _
    lax.fori_loop(0, (_CAP // 128) * (128 // _L), padp, 0)

    def group(g, M):
        off = jnp.minimum(g * 8, _ICOLS - 8)
        cps = [pltpu.async_copy(
                   itab_hbm.at[pl.ds(8 * i, 8), pl.ds(off * 128, 1024)],
                   sec.at[pl.ds(8 * i, 8), :], sem_s)
               for i in range(4)]
        for c in cps:
            c.wait()

        def match_t(t, mc):
            u = ilist[pl.ds(t * _L, _L)]
            pv = iplist[pl.ds(t * _L, _L)]
            c = lax.shift_right_logical(u, 7)
            m = (c >= off) & (c < off + 8)
            base = jnp.minimum(mc, _GCAP)
            plsc.store_compressed(tmpu.at[pl.ds(base, _L)], u, mask=m)
            plsc.store_compressed(tmpp.at[pl.ds(base, _L)], pv, mask=m)
            return mc + plsc.all_reduce_population_count(m)[0]

        mc = lax.fori_loop(0, _CAP // _L, match_t, jnp.int32(0))
        mc = jnp.minimum(mc, _GCAP)

        for mt in range(_GCAP // _L):
            mu = tmpu[pl.ds(mt * _L, _L)]
            mp = tmpp[pl.ds(mt * _L, _L)]
            mm = (mt * _L + iota) < mc
            colv = ((lax.shift_right_logical(mu, 7) - off) * 128
                    + (mu & 127)) & 1023
            mj = jnp.minimum(M + mt * _L + iota, _CAP - 1)
            for d in range(_D):
                dv = jnp.full((_L,), d, jnp.int32)
                val = plsc.load_gather(sec, [dv, colv], mask=mm)
                plsc.store_scatter(irow, [mj * _D + dv], val, mask=mm)
            plsc.store_scatter(
                plist2, [lax.shift_right_logical(mj, 7), mj & 127],
                mp, mask=mm)
        return jnp.minimum(M + mc, _CAP)

    M = lax.fori_loop(lo_g, hi_g, group, jnp.int32(0))

    def chunk(ch, carry):
        prow = plist2.at[ch]
        pltpu.async_copy(scr_hbm.at[prow], urows, sem_g).wait()
        for q in range(128 // _L):
            j = ch * 128 + q * _L + iota
            jm = j < M
            mrow = q * _L + iota
            acc = jnp.zeros((_L,), jnp.float32)
            for d in range(_D):
                dv = jnp.full((_L,), d, jnp.int32)
                uu = plsc.load_gather(urows, [mrow, dv])
                ii = plsc.load_gather(
                    irow, [jnp.minimum(j, _CAP - 1) * _D + dv])
                acc = acc + uu * ii
            plsc.store_scatter(outst, [mrow, jnp.zeros((_L,), jnp.int32)],
                               acc, mask=jm)
        pltpu.async_copy(outst, out_hbm.at[prow], sem_o).wait()
        return carry

    lax.fori_loop(0, _CAP // 128, chunk, 0)


@jax.jit
def kernel(user_ids, item_ids, user_table, item_table):
    mesh = plsc.VectorSubcoreMesh(core_axis_name="c", subcore_axis_name="s")
    params = pltpu.CompilerParams(
        needs_layout_passes=False, use_tc_tiling_on_sc=True,
        disable_bounds_checks=True,
    )
    k1 = pl.kernel(
        _u_body,
        out_type=jax.ShapeDtypeStruct((_B + 1, 128), jnp.float32),
        mesh=mesh,
        scratch_types=[
            pltpu.VMEM((2048,), jnp.int32),
            pltpu.VMEM((_CAP + _L,), jnp.int32),
            pltpu.VMEM((_CAP + _L,), jnp.int32),
            pltpu.VMEM((32, 1024), jnp.float32),
            pltpu.VMEM((_FL + _L,), jnp.int32),
            pltpu.VMEM((_FL + _L,), jnp.int32),
            pltpu.VMEM((_FL, 128), jnp.float32),
            pltpu.VMEM((_FL,), jnp.int32),
            pltpu.SemaphoreType.DMA,
            pltpu.SemaphoreType.DMA,
        ],
        compiler_params=params,
    )
    scratch = k1(user_ids, user_table.T)

    k2 = pl.kernel(
        _i_body,
        out_type=jax.ShapeDtypeStruct((_B + 1, 128), jnp.float32),
        mesh=mesh,
        scratch_types=[
            pltpu.VMEM((2048,), jnp.int32),
            pltpu.VMEM((_CAP + _L,), jnp.int32),
            pltpu.VMEM((_CAP + _L,), jnp.int32),
            pltpu.VMEM((32, 1024), jnp.float32),
            pltpu.VMEM((_GCAP + _L,), jnp.int32),
            pltpu.VMEM((_GCAP + _L,), jnp.int32),
            pltpu.VMEM((_CAP * _D,), jnp.float32),
            pltpu.VMEM((_CAP // 128, 128), jnp.int32),
            pltpu.VMEM((128, 128), jnp.float32),
            pltpu.VMEM((128, 128), jnp.float32),
            pltpu.SemaphoreType.DMA,
            pltpu.SemaphoreType.DMA,
            pltpu.SemaphoreType.DMA,
        ],
        compiler_params=params,
    )
    out2d = k2(item_ids, item_table.T, scratch)
    return out2d[:_B, 0]


# speedup vs baseline: 16.3255x; 16.3255x over previous
import functools

import jax
import jax.numpy as jnp
from jax import lax
from jax.experimental import pallas as pl
from jax.experimental.pallas import tpu as pltpu
from jax.experimental.pallas import tpu_sc as plsc

_NC = 2
_NS = 16
_NW = _NC * _NS
_L = 16
_D = 32
_B = 16384
_NSCR = _B + 128

_UCOLS = 7813
_ICOLS = 782
_UG = 489
_IG = 98

_CAP = 768
_GCAP = 288
_FL = 80


def _prescan(ids_hbm, idbuf, lst_id, lst_pos, lo_c, hi_c, pad_id):
    iota = lax.iota(jnp.int32, _L)
    npieces = _B // 2048

    def piece(p, cnt):
        pltpu.sync_copy(ids_hbm.at[pl.ds(p * 2048, 2048)], idbuf)

        def inner(t, cnt):
            u = idbuf[pl.ds(t * _L, _L)]
            c = lax.shift_right_logical(u, 7)
            m = (c >= lo_c) & (c < hi_c)
            pos = p * 2048 + t * _L + iota
            base = jnp.minimum(cnt, _CAP)
            plsc.store_compressed(lst_id.at[pl.ds(base, _L)], u, mask=m)
            plsc.store_compressed(lst_pos.at[pl.ds(base, _L)], pos, mask=m)
            return cnt + plsc.all_reduce_population_count(m)[0]

        return lax.fori_loop(0, 2048 // _L, inner, cnt)

    def padfill(t, _):
        lst_id[pl.ds(t * _L, _L)] = jnp.full((_L,), pad_id, jnp.int32)
        lst_pos[pl.ds(t * _L, _L)] = _B + ((t * _L) % 128) + lax.iota(jnp.int32, _L)
        return _

    lax.fori_loop(0, (_CAP + _L) // _L, padfill, 0)
    return lax.fori_loop(0, npieces, piece, jnp.int32(0))


def _u_body(uids_hbm, utab_hbm, scr_hbm,
            idbuf, ulist, uplist, sec, tmpu, tmpp, rows, posc,
            sem_s, sem_f):
    iota = lax.iota(jnp.int32, _L)
    wid = lax.axis_index("s") * _NC + lax.axis_index("c")
    lo_g = lax.shift_right_logical(_UG * wid, 5)
    hi_g = lax.shift_right_logical(_UG * (wid + 1), 5)

    cnt = _prescan(uids_hbm, idbuf, ulist, uplist,
                   lo_g * 16, hi_g * 16, jnp.int32(0x7FFFFFF))

    def group(g, carry):
        off = jnp.minimum(g * 16, _UCOLS - 16)
        cps = [pltpu.async_copy(
                   utab_hbm.at[pl.ds(8 * i, 8), pl.ds(off * 128, 2048)],
                   sec.at[pl.ds(8 * i, 8), :], sem_s)
               for i in range(4)]
        for c in cps:
            c.wait()

        def pad_t(t, _):
            tmpp[pl.ds(t * _L, _L)] = _B + ((t * _L) % 128) + iota
            return _
        lax.fori_loop(0, (_FL + _L) // _L, pad_t, 0)

        def match_t(t, mc):
            u = ulist[pl.ds(t * _L, _L)]
            pv = uplist[pl.ds(t * _L, _L)]
            c = lax.shift_right_logical(u, 7)
            m = (c >= off) & (c < off + 16)
            base = jnp.minimum(mc, _FL)
            plsc.store_compressed(tmpu.at[pl.ds(base, _L)], u, mask=m)
            plsc.store_compressed(tmpp.at[pl.ds(base, _L)], pv, mask=m)
            return mc + plsc.all_reduce_population_count(m)[0]

        mc = lax.fori_loop(0, _CAP // _L, match_t, jnp.int32(0))
        mc = jnp.minimum(mc, _FL)

        for mt in range(_FL // _L):
            mu = tmpu[pl.ds(mt * _L, _L)]
            mm = (mt * _L + iota) < mc
            colv = ((lax.shift_right_logical(mu, 7) - off) * 128
                    + (mu & 127)) & 2047
            mrow = mt * _L + iota
            for d in range(_D):
                dv = jnp.full((_L,), d, jnp.int32)
                val = plsc.load_gather(sec, [dv, colv], mask=mm)
                plsc.store_scatter(rows, [mrow, dv], val, mask=mm)

        def cp_t(t, _):
            posc[pl.ds(t * _L, _L)] = tmpp[pl.ds(t * _L, _L)]
            return _
        lax.fori_loop(0, _FL // _L, cp_t, 0)
        pltpu.async_copy(rows, scr_hbm.at[posc], sem_f).wait()
        return carry

    lax.fori_loop(lo_g, hi_g, group, 0)


def _i_body(iids_hbm, itab_hbm, scr_hbm, out_hbm,
            idbuf, ilist, iplist, sec, tmpu, tmpp, irow, plist2, urows,
            outst, sem_s, sem_g, sem_o):
    iota = lax.iota(jnp.int32, _L)
    wid = lax.axis_index("s") * _NC + lax.axis_index("c")
    lo_g = lax.shift_right_logical(_IG * wid, 5)
    hi_g = lax.shift_right_logical(_IG * (wid + 1), 5)

    cnt = _prescan(iids_hbm, idbuf, ilist, iplist,
                   lo_g * 8, hi_g * 8, jnp.int32(0x7FFFFFF))

    def padp(t, _):
        r = t // (128 // _L)
        s = (t % (128 // _L)) * _L
        plsc.store_scatter(
            plist2, [jnp.full((_L,), r, jnp.int32), s + iota],
            _B + s + iota)
        return _
    lax.fori_loop(0, (_CAP // 128) * (128 // _L), padp, 0)

    def group(g, M):
        off = jnp.minimum(g * 8, _ICOLS - 8)
        cps = [pltpu.async_copy(
                   itab_hbm.at[pl.ds(8 * i, 8), pl.ds(off * 128, 1024)],
                   sec.at[pl.ds(8 * i, 8), :], sem_s)
               for i in range(4)]
        for c in cps:
            c.wait()

        def match_t(t, mc):
            u = ilist[pl.ds(t * _L, _L)]
            pv = iplist[pl.ds(t * _L, _L)]
            c = lax.shift_right_logical(u, 7)
            m = (c >= off) & (c < off + 8)
            base = jnp.minimum(mc, _GCAP)
            plsc.store_compressed(tmpu.at[pl.ds(base, _L)], u, mask=m)
            plsc.store_compressed(tmpp.at[pl.ds(base, _L)], pv, mask=m)
            return mc + plsc.all_reduce_population_count(m)[0]

        mc = lax.fori_loop(0, _CAP // _L, match_t, jnp.int32(0))
        mc = jnp.minimum(mc, _GCAP)

        for mt in range(_GCAP // _L):
            mu = tmpu[pl.ds(mt * _L, _L)]
            mp = tmpp[pl.ds(mt * _L, _L)]
            mm = (mt * _L + iota) < mc
            colv = ((lax.shift_right_logical(mu, 7) - off) * 128
                    + (mu & 127)) & 1023
            mj = jnp.minimum(M + mt * _L + iota, _CAP - 1)
            for d in range(_D):
                dv = jnp.full((_L,), d, jnp.int32)
                val = plsc.load_gather(sec, [dv, colv], mask=mm)
                plsc.store_scatter(irow, [mj * _D + dv], val, mask=mm)
            plsc.store_scatter(
                plist2, [lax.shift_right_logical(mj, 7), mj & 127],
                mp, mask=mm)
        return jnp.minimum(M + mc, _CAP)

    M = lax.fori_loop(lo_g, hi_g, group, jnp.int32(0))

    def chunk(ch, carry):
        prow = plist2.at[ch]
        pltpu.async_copy(scr_hbm.at[prow], urows, sem_g).wait()
        for q in range(128 // _L):
            j = ch * 128 + q * _L + iota
            jm = j < M
            mrow = q * _L + iota
            acc = jnp.zeros((_L,), jnp.float32)
            for d in range(_D):
                dv = jnp.full((_L,), d, jnp.int32)
                uu = plsc.load_gather(urows, [mrow, dv])
                ii = plsc.load_gather(
                    irow, [jnp.minimum(j, _CAP - 1) * _D + dv])
                acc = acc + uu * ii
            plsc.store_scatter(outst, [mrow, jnp.zeros((_L,), jnp.int32)],
                               acc, mask=jm)
        pltpu.async_copy(outst, out_hbm.at[prow], sem_o).wait()
        return carry

    lax.fori_loop(0, _CAP // 128, chunk, 0)


@jax.jit
def kernel(user_ids, item_ids, user_table, item_table):
    mesh = plsc.VectorSubcoreMesh(core_axis_name="c", subcore_axis_name="s")
    params = pltpu.CompilerParams(
        needs_layout_passes=False, use_tc_tiling_on_sc=True,
        disable_bounds_checks=True,
    )
    k1 = pl.kernel(
        _u_body,
        out_type=jax.ShapeDtypeStruct((_NSCR, 128), jnp.float32),
        mesh=mesh,
        scratch_types=[
            pltpu.VMEM((2048,), jnp.int32),
            pltpu.VMEM((_CAP + _L,), jnp.int32),
            pltpu.VMEM((_CAP + _L,), jnp.int32),
            pltpu.VMEM((32, 2048), jnp.float32),
            pltpu.VMEM((_FL + _L,), jnp.int32),
            pltpu.VMEM((_FL + _L,), jnp.int32),
            pltpu.VMEM((_FL, 128), jnp.float32),
            pltpu.VMEM((_FL,), jnp.int32),
            pltpu.SemaphoreType.DMA,
            pltpu.SemaphoreType.DMA,
        ],
        compiler_params=params,
    )
    scratch = k1(user_ids, user_table.T)

    k2 = pl.kernel(
        _i_body,
        out_type=jax.ShapeDtypeStruct((_NSCR, 128), jnp.float32),
        mesh=mesh,
        scratch_types=[
            pltpu.VMEM((2048,), jnp.int32),
            pltpu.VMEM((_CAP + _L,), jnp.int32),
            pltpu.VMEM((_CAP + _L,), jnp.int32),
            pltpu.VMEM((32, 1024), jnp.float32),
            pltpu.VMEM((_GCAP + _L,), jnp.int32),
            pltpu.VMEM((_GCAP + _L,), jnp.int32),
            pltpu.VMEM((_CAP * _D,), jnp.float32),
            pltpu.VMEM((_CAP // 128, 128), jnp.int32),
            pltpu.VMEM((128, 128), jnp.float32),
            pltpu.VMEM((128, 128), jnp.float32),
            pltpu.SemaphoreType.DMA,
            pltpu.SemaphoreType.DMA,
            pltpu.SemaphoreType.DMA,
        ],
        compiler_params=params,
    )
    out2d = k2(item_ids, item_table.T, scratch)
    return out2d[:_B, 0]
